# baseline (speedup 1.0000x reference)
import functools
import math

import jax
import jax.numpy as jnp
from jax import lax
from jax.experimental import pallas as pl
from jax.experimental.pallas import tpu as pltpu
from jax.experimental.pallas import tpu_sc as plsc

_LANES = 16


def kernel(x, table):
    B, S = x.shape
    V, D = table.shape
    N = B * S
    scale = jnp.float32(math.sqrt(D))

    idx = x.reshape(N).astype(jnp.int32)

    info = plsc.get_sparse_core_info()
    NC, NS = info.num_cores, info.num_subcores
    NW = NC * NS
    n_per_w = N // NW
    b_per_w = B // NW
    RB = 2
    C = RB * S
    n_chunks = n_per_w // C
    NBUF = 3
    U = 8

    mesh = plsc.VectorSubcoreMesh(core_axis_name="c", subcore_axis_name="s")

    @functools.partial(
        pl.kernel,
        mesh=mesh,
        compiler_params=pltpu.CompilerParams(use_tc_tiling_on_sc=False),
        out_type=jax.ShapeDtypeStruct((B, S, D), jnp.float32),
        scratch_types=[
            pltpu.VMEM((n_per_w,), jnp.int32),
            [pltpu.VMEM((C, D), jnp.float32)] * NBUF,
            [pltpu.SemaphoreType.DMA] * NBUF,
            [pltpu.SemaphoreType.DMA] * NBUF,
        ],
    )
    def lookup(idx_hbm, table_hbm, out_hbm, idx_all, rows, gsems, ssems):
        wid = lax.axis_index("s") * NC + lax.axis_index("c")
        base = wid * n_per_w
        base_b = wid * b_per_w

        pltpu.sync_copy(idx_hbm.at[pl.ds(base, n_per_w)], idx_all)

        def idx_slice(g):
            return idx_all.at[pl.ds(g * C, C)]

        def fire_gather(g, b):
            pltpu.async_copy(table_hbm.at[idx_slice(g)], rows[b], gsems[b])

        def wait_gather(g, b):
            pltpu.make_async_copy(
                table_hbm.at[idx_slice(g)], rows[b], gsems[b]
            ).wait()

        def fire_store(g, b):
            for k in range(RB):
                pltpu.async_copy(
                    rows[b].at[pl.ds(k * S, S)],
                    out_hbm.at[base_b + g * RB + k],
                    ssems[b],
                )

        def wait_store(g, b):
            for k in range(RB):
                pltpu.make_async_copy(
                    rows[b].at[pl.ds(k * S, S)],
                    out_hbm.at[base_b + g * RB + k],
                    ssems[b],
                ).wait()

        fire_gather(0, 0)
        fire_gather(1, 1)

        wait_gather(0, 0)
        fire_store(0, 0)
        fire_gather(2, 2)

        def steady(p, carry):
            for j in range(NBUF):
                g = 1 + p * NBUF + j
                b = (1 + j) % NBUF
                b2 = j % NBUF
                wait_gather(g, b)
                fire_store(g, b)
                wait_store(g - 1, b2)
                fire_gather(g + 2, b2)
            return carry

        n_steady = (n_chunks - 5) // NBUF
        lax.fori_loop(0, n_steady, steady, 0)

        for g in range(1 + n_steady * NBUF, n_chunks):
            b = g % NBUF
            b2 = (g + 2) % NBUF
            wait_gather(g, b)
            fire_store(g, b)
            if g + 2 < n_chunks:
                wait_store(g - 1, b2)
                fire_gather(g + 2, b2)

        for g in range(n_chunks - NBUF, n_chunks):
            wait_store(g, g % NBUF)

    out = lookup(idx, table)
    return out * scale


# speedup vs baseline: 1.2113x; 1.2113x over previous
import functools
import math

import jax
import jax.numpy as jnp
from jax import lax
from jax.experimental import pallas as pl
from jax.experimental.pallas import tpu as pltpu
from jax.experimental.pallas import tpu_sc as plsc

_LANES = 16


def kernel(x, table):
    B, S = x.shape
    V, D = table.shape
    N = B * S
    scale = jnp.float32(math.sqrt(D))

    idx = x.reshape(N).astype(jnp.int32)

    info = plsc.get_sparse_core_info()
    NC, NS = info.num_cores, info.num_subcores
    NW = NC * NS
    n_per_w = N // NW
    b_per_w = B // NW
    RB = 2
    C = RB * S
    n_chunks = n_per_w // C
    NBUF = 3
    U = 8

    mesh = plsc.VectorSubcoreMesh(core_axis_name="c", subcore_axis_name="s")

    @functools.partial(
        pl.kernel,
        mesh=mesh,
        compiler_params=pltpu.CompilerParams(use_tc_tiling_on_sc=False),
        out_type=jax.ShapeDtypeStruct((B, S, D), jnp.float32),
        scratch_types=[
            pltpu.VMEM((n_per_w,), jnp.int32),
            [pltpu.VMEM((C, D), jnp.float32)] * NBUF,
            [pltpu.SemaphoreType.DMA] * NBUF,
            [pltpu.SemaphoreType.DMA] * NBUF,
        ],
    )
    def lookup(idx_hbm, table_hbm, out_hbm, idx_all, rows, gsems, ssems):
        wid = lax.axis_index("s") * NC + lax.axis_index("c")
        base = wid * n_per_w
        base_b = wid * b_per_w

        pltpu.sync_copy(idx_hbm.at[pl.ds(base, n_per_w)], idx_all)

        def idx_slice(g):
            return idx_all.at[pl.ds(g * C, C)]

        def fire_gather(g, b):
            pltpu.async_copy(table_hbm.at[idx_slice(g)], rows[b], gsems[b])

        def wait_gather(g, b):
            pltpu.make_async_copy(
                table_hbm.at[idx_slice(g)], rows[b], gsems[b]
            ).wait()

        def scale_buf(b):
            r = rows[b]

            def body(i, c):
                for u in range(U):
                    for j in range(D // _LANES):
                        sl = pl.ds(j * _LANES, _LANES)
                        r[i * U + u, sl] = r[i * U + u, sl] * scale
                return c

            lax.fori_loop(0, C // U, body, 0)

        def fire_store(g, b):
            for k in range(RB):
                pltpu.async_copy(
                    rows[b].at[pl.ds(k * S, S)],
                    out_hbm.at[base_b + g * RB + k],
                    ssems[b],
                )

        def wait_store(g, b):
            for k in range(RB):
                pltpu.make_async_copy(
                    rows[b].at[pl.ds(k * S, S)],
                    out_hbm.at[base_b + g * RB + k],
                    ssems[b],
                ).wait()

        fire_gather(0, 0)
        fire_gather(1, 1)

        wait_gather(0, 0)
        scale_buf(0)
        fire_store(0, 0)
        fire_gather(2, 2)

        def steady(p, carry):
            for j in range(NBUF):
                g = 1 + p * NBUF + j
                b = (1 + j) % NBUF
                b2 = j % NBUF
                wait_gather(g, b)
                scale_buf(b)
                fire_store(g, b)
                wait_store(g - 1, b2)
                fire_gather(g + 2, b2)
            return carry

        n_steady = (n_chunks - 5) // NBUF
        lax.fori_loop(0, n_steady, steady, 0)

        for g in range(1 + n_steady * NBUF, n_chunks):
            b = g % NBUF
            b2 = (g + 2) % NBUF
            wait_gather(g, b)
            scale_buf(b)
            fire_store(g, b)
            if g + 2 < n_chunks:
                wait_store(g - 1, b2)
                fire_gather(g + 2, b2)

        for g in range(n_chunks - NBUF, n_chunks):
            wait_store(g, g % NBUF)

    out = lookup(idx, table)
    return out
